# baseline (speedup 1.0000x reference)
import jax
import jax.numpy as jnp
from jax import lax
from jax.experimental import pallas as pl
from jax.experimental.pallas import tpu as pltpu
from jax.experimental.pallas import tpu_sc as plsc

_NUM_CORES = 2
_NUM_SUBCORES = 16
_NUM_WORKERS = _NUM_CORES * _NUM_SUBCORES
_WINDOW = 800


def kernel(input_ids, table):
    batch, seq = input_ids.shape
    vocab, hidden = table.shape
    n = batch * seq
    ids = input_ids.reshape(n).astype(jnp.int32)
    tab_pad = jnp.pad(table, ((0, 0), (0, 128 - hidden)))

    per_worker = n // _NUM_WORKERS
    steps = per_worker // _WINDOW
    assert per_worker % _WINDOW == 0 and n % _NUM_WORKERS == 0

    mesh = plsc.VectorSubcoreMesh(core_axis_name="c", subcore_axis_name="s")

    @pl.kernel(
        out_type=jax.ShapeDtypeStruct((n, 128), table.dtype),
        mesh=mesh,
        scratch_types=[
            pltpu.VMEM((_WINDOW,), jnp.int32),
            pltpu.VMEM((_WINDOW, 128), table.dtype),
            pltpu.SemaphoreType.DMA,
        ],
    )
    def gather_kernel(tab_hbm, ids_hbm, out_hbm, idx_v, rows_v, sem):
        wid = lax.axis_index("s") * _NUM_CORES + lax.axis_index("c")
        base = wid * per_worker

        @pl.loop(0, steps)
        def _(s):
            off = base + s * _WINDOW
            pltpu.sync_copy(ids_hbm.at[pl.ds(off, _WINDOW)], idx_v)
            pltpu.async_copy(tab_hbm.at[idx_v], rows_v, sem).wait()
            pltpu.sync_copy(rows_v, out_hbm.at[pl.ds(off, _WINDOW)])

    out128 = gather_kernel(tab_pad, ids)
    return out128[:, :hidden].reshape(batch, seq, hidden)
